# baseline (speedup 1.0000x reference)
import functools

import jax
import jax.numpy as jnp
from jax import lax
from jax.experimental import pallas as pl
from jax.experimental.pallas import tpu as pltpu
from jax.experimental.pallas import tpu_sc as plsc

NS = 16
W_EDGES = 64
NBUF = 5
PAD_SPREAD = 8


def _pad_edges(src, dst, n_chunks, n):
    e = src.shape[0]
    per = e // n_chunks
    padded = -(-per // (NBUF * W_EDGES)) * (NBUF * W_EDGES)
    pad = padded - per
    pad_src = (jnp.arange(pad, dtype=jnp.int32) * 97) % n
    pad_dst = n + (jnp.arange(pad, dtype=jnp.int32) % PAD_SPREAD)
    srcp = jnp.concatenate(
        [src.reshape(n_chunks, per),
         jnp.broadcast_to(pad_src, (n_chunks, pad))], axis=1)
    dstp = jnp.concatenate(
        [dst.reshape(n_chunks, per),
         jnp.broadcast_to(pad_dst, (n_chunks, pad))], axis=1)
    n_ss = padded // (NBUF * W_EDGES)
    s4 = srcp.reshape(n_chunks, n_ss, NBUF, 1, W_EDGES)
    d4 = dstp.reshape(n_chunks, n_ss, NBUF, 1, W_EDGES)
    idx3 = jnp.concatenate([s4, d4], axis=3).reshape(
        n_chunks * n_ss, NBUF * 2, W_EDGES)
    return idx3, padded, n_ss


def _edge_pipeline(h_ref, acc, idx3_ref, idx_row_base,
                   idxb, rows, gsem, ssem, isem, n_ss):
    def idx_args(ss, p):
        return (idx3_ref.at[idx_row_base + ss], idxb.at[p], isem.at[p])

    def g_args(p, b):
        return (h_ref.at[idxb.at[p].at[2 * b]], rows.at[b], gsem.at[b])

    def s_args(p, b):
        return (rows.at[b], acc.at[idxb.at[p].at[2 * b + 1]], ssem.at[b])

    def do_ss(ss, p, prefetch_ss):
        if prefetch_ss is not None:
            pltpu.async_copy(*idx_args(prefetch_ss, 1 - p))
        for b in range(NBUF):
            pltpu.make_async_copy(*g_args(p, b)).wait()
            pltpu.async_copy(*s_args(p, b), add=True)
        if prefetch_ss is not None:
            pltpu.make_async_copy(*idx_args(prefetch_ss, 1 - p)).wait()
        for b in range(NBUF):
            pltpu.make_async_copy(*s_args(p, b)).wait()
            if prefetch_ss is not None:
                pltpu.async_copy(*g_args(1 - p, b))

    pro = idx_args(0, 0)
    pltpu.sync_copy(pro[0], pro[1])
    for b in range(NBUF):
        pltpu.async_copy(*g_args(0, b))

    n_double = (n_ss - 1) // 2

    def body(i2, carry):
        do_ss(2 * i2, 0, 2 * i2 + 1)
        do_ss(2 * i2 + 1, 1, 2 * i2 + 2)
        return carry

    lax.fori_loop(0, n_double, body, 0)
    for ss in range(2 * n_double, n_ss):
        do_ss(ss, ss % 2, ss + 1 if ss < n_ss - 1 else None)

def _seed(h_ref, acc, s, seed_rows, rem_base, rem_rows):
    pltpu.sync_copy(h_ref.at[pl.ds(s * seed_rows, seed_rows)],
                    acc.at[pl.ds(s * seed_rows, seed_rows)])

    @pl.when(s == NS - 1)
    def _():
        pltpu.sync_copy(h_ref.at[pl.ds(rem_base, rem_rows)],
                        acc.at[pl.ds(rem_base, rem_rows)])


def _writeback(acc, out_ref, s, seed_rows, rem_base, rem_rows):
    pltpu.sync_copy(acc.at[pl.ds(s * seed_rows, seed_rows)],
                    out_ref.at[pl.ds(s * seed_rows, seed_rows)])

    @pl.when(s == NS - 1)
    def _():
        pltpu.sync_copy(acc.at[pl.ds(rem_base, rem_rows)],
                        out_ref.at[pl.ds(rem_base, rem_rows)])


def _sc_scratch(n, fw, e_per_tile):
    return (
        pltpu.VMEM_SHARED((n + PAD_SPREAD, fw), jnp.float32),
        pltpu.VMEM((2, NBUF * 2, W_EDGES), jnp.int32),
        pltpu.VMEM((NBUF, W_EDGES, fw), jnp.float32),
        pltpu.SemaphoreType.DMA((NBUF,)),
        pltpu.SemaphoreType.DMA((NBUF,)),
        pltpu.SemaphoreType.DMA((2,)),
    )


def _sc_aggregate_edges(h, idx3, e_per_tile, n_ss):
    n, fw = h.shape
    seed_rows = (n // NS) // 8 * 8
    rem_base = seed_rows * NS
    rem_rows = n - rem_base

    mesh = plsc.VectorSubcoreMesh(core_axis_name="c", subcore_axis_name="s")

    @functools.partial(
        pl.kernel,
        out_type=(
            jax.ShapeDtypeStruct((n, fw), jnp.float32),
            jax.ShapeDtypeStruct((n, fw), jnp.float32),
        ),
        mesh=mesh,
        scratch_types=_sc_scratch(n, fw, e_per_tile),
    )
    def agg(h_ref, idx3_ref, o0_ref, o1_ref, acc, idxb, rows,
            gsem, ssem, isem):
        c = lax.axis_index("c")
        s = lax.axis_index("s")
        tid = c * NS + s

        def run(out_ref):
            _seed(h_ref, acc, s, seed_rows, rem_base, rem_rows)
            plsc.subcore_barrier()
            _edge_pipeline(h_ref, acc, idx3_ref,
                           tid * n_ss, idxb, rows, gsem, ssem, isem, n_ss)
            plsc.subcore_barrier()
            _writeback(acc, out_ref, s, seed_rows, rem_base, rem_rows)

        @pl.when(c == 0)
        def _():
            run(o0_ref)

        @pl.when(c == 1)
        def _():
            run(o1_ref)

    return agg(h, idx3)


def _sc_aggregate(h_lo, h_hi, idx3, e_per_tile, n_ss):
    n, fw = h_lo.shape
    seed_rows = (n // NS) // 8 * 8
    rem_base = seed_rows * NS
    rem_rows = n - rem_base

    mesh = plsc.VectorSubcoreMesh(core_axis_name="c", subcore_axis_name="s")

    @functools.partial(
        pl.kernel,
        out_type=(
            jax.ShapeDtypeStruct((n, fw), jnp.float32),
            jax.ShapeDtypeStruct((n, fw), jnp.float32),
        ),
        mesh=mesh,
        scratch_types=_sc_scratch(n, fw, e_per_tile),
    )
    def agg(h_lo_ref, h_hi_ref, idx3_ref, out_lo_ref, out_hi_ref,
            acc, idxb, rows, gsem, ssem, isem):
        c = lax.axis_index("c")
        s = lax.axis_index("s")

        def run(h_ref, out_ref):
            _seed(h_ref, acc, s, seed_rows, rem_base, rem_rows)
            plsc.subcore_barrier()
            _edge_pipeline(h_ref, acc, idx3_ref,
                           s * n_ss, idxb, rows, gsem, ssem, isem, n_ss)
            plsc.subcore_barrier()
            _writeback(acc, out_ref, s, seed_rows, rem_base, rem_rows)

        @pl.when(c == 0)
        def _():
            run(h_lo_ref, out_lo_ref)

        @pl.when(c == 1)
        def _():
            run(h_hi_ref, out_hi_ref)

    return agg(h_lo, h_hi, idx3)


_ROWS = 2000


def _mlp0(o0, o1, x, W1, b1, W2, b2):
    n, fw = o0.shape
    h = W2.shape[0]

    def body(o0_ref, o1_ref, x_ref, w1_ref, b1_ref, w2_ref, b2_ref,
             out_lo_ref, out_hi_ref):
        hs = o0_ref[...] + o1_ref[...] - x_ref[...]
        z = jnp.dot(hs, w1_ref[...], preferred_element_type=jnp.float32)
        z = jnp.maximum(z + b1_ref[...], 0.0)
        z = jnp.dot(z, w2_ref[...], preferred_element_type=jnp.float32) + b2_ref[...]
        z = jnp.maximum(z, 0.0)
        hw = h // 2
        out_lo_ref[...] = z[:, :hw]
        out_hi_ref[...] = z[:, hw:]

    grid = (n // _ROWS,)
    return pl.pallas_call(
        body,
        grid=grid,
        in_specs=[
            pl.BlockSpec((_ROWS, fw), lambda i: (i, 0)),
            pl.BlockSpec((_ROWS, fw), lambda i: (i, 0)),
            pl.BlockSpec((_ROWS, fw), lambda i: (i, 0)),
            pl.BlockSpec((fw, h), lambda i: (0, 0)),
            pl.BlockSpec((1, h), lambda i: (0, 0)),
            pl.BlockSpec((h, h), lambda i: (0, 0)),
            pl.BlockSpec((1, h), lambda i: (0, 0)),
        ],
        out_specs=[
            pl.BlockSpec((_ROWS, h // 2), lambda i: (i, 0)),
            pl.BlockSpec((_ROWS, h // 2), lambda i: (i, 0)),
        ],
        out_shape=[
            jax.ShapeDtypeStruct((n, h // 2), jnp.float32),
            jax.ShapeDtypeStruct((n, h // 2), jnp.float32),
        ],
    )(o0, o1, x, W1, b1, W2, b2)


def _mlp_res(hs_lo, hs_hi, hin_lo, hin_hi, W1a, W1b, b1, W2, b2):
    n, fw = hs_lo.shape
    h = W2.shape[0]

    def body(lo_ref, hi_ref, ilo_ref, ihi_ref, w1a_ref, w1b_ref, b1_ref,
             w2_ref, b2_ref, out_lo_ref, out_hi_ref):
        z = jnp.dot(lo_ref[...], w1a_ref[...], preferred_element_type=jnp.float32)
        z += jnp.dot(hi_ref[...], w1b_ref[...], preferred_element_type=jnp.float32)
        z = jnp.maximum(z + b1_ref[...], 0.0)
        z = jnp.dot(z, w2_ref[...], preferred_element_type=jnp.float32) + b2_ref[...]
        hw = h // 2
        out_lo_ref[...] = jnp.maximum(z[:, :hw] + lo_ref[...], 0.0) + ilo_ref[...]
        out_hi_ref[...] = jnp.maximum(z[:, hw:] + hi_ref[...], 0.0) + ihi_ref[...]

    grid = (n // _ROWS,)
    return pl.pallas_call(
        body,
        grid=grid,
        in_specs=[
            pl.BlockSpec((_ROWS, fw), lambda i: (i, 0)),
            pl.BlockSpec((_ROWS, fw), lambda i: (i, 0)),
            pl.BlockSpec((_ROWS, fw), lambda i: (i, 0)),
            pl.BlockSpec((_ROWS, fw), lambda i: (i, 0)),
            pl.BlockSpec((fw, h), lambda i: (0, 0)),
            pl.BlockSpec((fw, h), lambda i: (0, 0)),
            pl.BlockSpec((1, h), lambda i: (0, 0)),
            pl.BlockSpec((h, h), lambda i: (0, 0)),
            pl.BlockSpec((1, h), lambda i: (0, 0)),
        ],
        out_specs=[
            pl.BlockSpec((_ROWS, fw), lambda i: (i, 0)),
            pl.BlockSpec((_ROWS, fw), lambda i: (i, 0)),
        ],
        out_shape=[
            jax.ShapeDtypeStruct((n, fw), jnp.float32),
            jax.ShapeDtypeStruct((n, fw), jnp.float32),
        ],
    )(hs_lo, hs_hi, hin_lo, hin_hi, W1a, W1b, b1, W2, b2)


def _mlp_pool(hs_lo, hs_hi, hin_lo, hin_hi, batch2d, W1a, W1b, b1, W2, b2, g):
    n, fw = hs_lo.shape
    h = W2.shape[0]

    def body(lo_ref, hi_ref, ilo_ref, ihi_ref, b_ref, w1a_ref, w1b_ref, b1_ref,
             w2_ref, b2_ref, sums_ref, counts_ref):
        z = jnp.dot(lo_ref[...], w1a_ref[...], preferred_element_type=jnp.float32)
        z += jnp.dot(hi_ref[...], w1b_ref[...], preferred_element_type=jnp.float32)
        z = jnp.maximum(z + b1_ref[...], 0.0)
        z = jnp.dot(z, w2_ref[...], preferred_element_type=jnp.float32) + b2_ref[...]
        hw = h // 2
        o_lo = jnp.maximum(z[:, :hw] + lo_ref[...], 0.0) + ilo_ref[...]
        o_hi = jnp.maximum(z[:, hw:] + hi_ref[...], 0.0) + ihi_ref[...]
        h4 = jnp.concatenate([o_lo, o_hi], axis=1)
        onehot = (b_ref[...] == lax.broadcasted_iota(jnp.int32, (_ROWS, g), 1))
        onehot = onehot.astype(jnp.float32)
        psum = lax.dot_general(onehot, h4, (((0,), (0,)), ((), ())),
                               preferred_element_type=jnp.float32)
        ones = jnp.ones((_ROWS, 8), jnp.float32)
        pcnt = lax.dot_general(onehot, ones, (((0,), (0,)), ((), ())),
                               preferred_element_type=jnp.float32)

        @pl.when(pl.program_id(0) == 0)
        def _():
            sums_ref[...] = jnp.zeros_like(sums_ref)
            counts_ref[...] = jnp.zeros_like(counts_ref)

        sums_ref[...] += psum
        counts_ref[...] += pcnt

    grid = (n // _ROWS,)
    return pl.pallas_call(
        body,
        grid=grid,
        in_specs=[
            pl.BlockSpec((_ROWS, fw), lambda i: (i, 0)),
            pl.BlockSpec((_ROWS, fw), lambda i: (i, 0)),
            pl.BlockSpec((_ROWS, fw), lambda i: (i, 0)),
            pl.BlockSpec((_ROWS, fw), lambda i: (i, 0)),
            pl.BlockSpec((_ROWS, 1), lambda i: (i, 0)),
            pl.BlockSpec((fw, h), lambda i: (0, 0)),
            pl.BlockSpec((fw, h), lambda i: (0, 0)),
            pl.BlockSpec((1, h), lambda i: (0, 0)),
            pl.BlockSpec((h, h), lambda i: (0, 0)),
            pl.BlockSpec((1, h), lambda i: (0, 0)),
        ],
        out_specs=[
            pl.BlockSpec((g, h), lambda i: (0, 0)),
            pl.BlockSpec((g, 8), lambda i: (0, 0)),
        ],
        out_shape=[
            jax.ShapeDtypeStruct((g, h), jnp.float32),
            jax.ShapeDtypeStruct((g, 8), jnp.float32),
        ],
    )(hs_lo, hs_hi, hin_lo, hin_hi, batch2d, W1a, W1b, b1, W2, b2)


def _classifier(sums, counts, Wc1, bc1, Wc2, bc2):
    g, h = sums.shape
    c = Wc2.shape[1]

    def body(sums_ref, counts_ref, w1_ref, b1_ref, w2_ref, b2_ref, out_ref):
        cnt = jnp.maximum(counts_ref[...][:, :1], 1.0)
        pooled = sums_ref[...] / cnt
        z = jnp.maximum(
            jnp.dot(pooled, w1_ref[...], preferred_element_type=jnp.float32)
            + b1_ref[...], 0.0)
        out_ref[...] = (
            jnp.dot(z, w2_ref[...], preferred_element_type=jnp.float32)
            + b2_ref[...])

    return pl.pallas_call(
        body,
        out_shape=jax.ShapeDtypeStruct((g, c), jnp.float32),
    )(sums, counts, Wc1, bc1, Wc2, bc2)


def kernel(x, edge_index, batch, W1_0, b1_0, W2_0, b2_0, Ws1, bs1, Ws2, bs2,
           Wc1, bc1, Wc2, bc2):
    n, d_in = x.shape
    h = W2_0.shape[0]
    g = 64
    src = edge_index[0]
    dst = edge_index[1]
    hw = h // 2

    idx30, ept0, nss0 = _pad_edges(src, dst, 2 * NS, n)
    idx3f, eptf, nssf = _pad_edges(src, dst, NS, n)

    o0, o1 = _sc_aggregate_edges(x, idx30, ept0, nss0)
    h_lo, h_hi = _mlp0(o0, o1, x, W1_0, b1_0.reshape(1, -1), W2_0,
                       b2_0.reshape(1, -1))

    for i in range(Ws1.shape[0]):
        hs_lo, hs_hi = _sc_aggregate(h_lo, h_hi, idx3f, eptf, nssf)
        W1a, W1b = Ws1[i][:hw], Ws1[i][hw:]
        b1i = bs1[i].reshape(1, -1)
        W2i, b2i = Ws2[i], bs2[i].reshape(1, -1)
        if i < Ws1.shape[0] - 1:
            h_lo, h_hi = _mlp_res(hs_lo, hs_hi, h_lo, h_hi, W1a, W1b, b1i,
                                  W2i, b2i)
        else:
            sums, counts = _mlp_pool(hs_lo, hs_hi, h_lo, h_hi,
                                     batch.reshape(n, 1), W1a, W1b, b1i,
                                     W2i, b2i, g)

    return _classifier(sums, counts, Wc1, bc1.reshape(1, -1), Wc2,
                       bc2.reshape(1, -1))


# speedup vs baseline: 1.0034x; 1.0034x over previous
import functools

import jax
import jax.numpy as jnp
from jax import lax
from jax.experimental import pallas as pl
from jax.experimental.pallas import tpu as pltpu
from jax.experimental.pallas import tpu_sc as plsc

NS = 16
W_EDGES = 64
NBUF = 5
PAD_SPREAD = 8


def _pad_edges(src, dst, n_chunks, n):
    e = src.shape[0]
    per = e // n_chunks
    padded = -(-per // (NBUF * W_EDGES)) * (NBUF * W_EDGES)
    pad = padded - per
    pad_src = (jnp.arange(pad, dtype=jnp.int32) * 97) % n
    pad_dst = n + (jnp.arange(pad, dtype=jnp.int32) % PAD_SPREAD)
    srcp = jnp.concatenate(
        [src.reshape(n_chunks, per),
         jnp.broadcast_to(pad_src, (n_chunks, pad))], axis=1)
    dstp = jnp.concatenate(
        [dst.reshape(n_chunks, per),
         jnp.broadcast_to(pad_dst, (n_chunks, pad))], axis=1)
    n_ss = padded // (NBUF * W_EDGES)
    s4 = srcp.reshape(n_chunks, n_ss, NBUF, 1, W_EDGES)
    d4 = dstp.reshape(n_chunks, n_ss, NBUF, 1, W_EDGES)
    idx3 = jnp.concatenate([s4, d4], axis=3).reshape(
        n_chunks * n_ss, NBUF * 2, W_EDGES)
    return idx3, padded, n_ss


def _edge_pipeline(h_ref, acc, idx3_ref, idx_row_base,
                   idxb, rows, gsem, ssem, isem, n_ss):
    def idx_args(ss, p):
        return (idx3_ref.at[idx_row_base + ss], idxb.at[p], isem.at[p])

    def g_args(p, b):
        return (h_ref.at[idxb.at[p].at[2 * b]], rows.at[b], gsem.at[b])

    def s_args(p, b):
        return (rows.at[b], acc.at[idxb.at[p].at[2 * b + 1]], ssem.at[b])

    def do_ss(ss, p, prefetch_ss):
        if prefetch_ss is not None:
            pltpu.async_copy(*idx_args(prefetch_ss, 1 - p))
        for b in range(NBUF):
            pltpu.make_async_copy(*g_args(p, b)).wait()
            pltpu.async_copy(*s_args(p, b), add=True)
        if prefetch_ss is not None:
            pltpu.make_async_copy(*idx_args(prefetch_ss, 1 - p)).wait()
        for b in range(NBUF):
            pltpu.make_async_copy(*s_args(p, b)).wait()
            if prefetch_ss is not None:
                pltpu.async_copy(*g_args(1 - p, b))

    pro = idx_args(0, 0)
    pltpu.sync_copy(pro[0], pro[1])
    for b in range(NBUF):
        pltpu.async_copy(*g_args(0, b))

    n_double = (n_ss - 1) // 2

    def body(i2, carry):
        do_ss(2 * i2, 0, 2 * i2 + 1)
        do_ss(2 * i2 + 1, 1, 2 * i2 + 2)
        return carry

    lax.fori_loop(0, n_double, body, 0)
    for ss in range(2 * n_double, n_ss):
        do_ss(ss, ss % 2, ss + 1 if ss < n_ss - 1 else None)

def _seed(h_ref, acc, s, seed_rows, rem_base, rem_rows):
    pltpu.sync_copy(h_ref.at[pl.ds(s * seed_rows, seed_rows)],
                    acc.at[pl.ds(s * seed_rows, seed_rows)])

    @pl.when(s == NS - 1)
    def _():
        pltpu.sync_copy(h_ref.at[pl.ds(rem_base, rem_rows)],
                        acc.at[pl.ds(rem_base, rem_rows)])


def _writeback(acc, out_ref, s, seed_rows, rem_base, rem_rows):
    pltpu.sync_copy(acc.at[pl.ds(s * seed_rows, seed_rows)],
                    out_ref.at[pl.ds(s * seed_rows, seed_rows)])

    @pl.when(s == NS - 1)
    def _():
        pltpu.sync_copy(acc.at[pl.ds(rem_base, rem_rows)],
                        out_ref.at[pl.ds(rem_base, rem_rows)])


def _sc_scratch(n, fw, e_per_tile):
    return (
        pltpu.VMEM_SHARED((n + PAD_SPREAD, fw), jnp.float32),
        pltpu.VMEM((2, NBUF * 2, W_EDGES), jnp.int32),
        pltpu.VMEM((NBUF, W_EDGES, fw), jnp.float32),
        pltpu.SemaphoreType.DMA((NBUF,)),
        pltpu.SemaphoreType.DMA((NBUF,)),
        pltpu.SemaphoreType.DMA((2,)),
    )


def _sc_aggregate_edges(h, idx3, e_per_tile, n_ss):
    n, fw = h.shape
    seed_rows = (n // NS) // 8 * 8
    rem_base = seed_rows * NS
    rem_rows = n - rem_base

    mesh = plsc.VectorSubcoreMesh(core_axis_name="c", subcore_axis_name="s")

    @functools.partial(
        pl.kernel,
        out_type=(
            jax.ShapeDtypeStruct((n, fw), jnp.float32),
            jax.ShapeDtypeStruct((n, fw), jnp.float32),
        ),
        mesh=mesh,
        scratch_types=_sc_scratch(n, fw, e_per_tile),
    )
    def agg(h_ref, idx3_ref, o0_ref, o1_ref, acc, idxb, rows,
            gsem, ssem, isem):
        c = lax.axis_index("c")
        s = lax.axis_index("s")
        tid = c * NS + s

        def run(out_ref):
            _seed(h_ref, acc, s, seed_rows, rem_base, rem_rows)
            plsc.subcore_barrier()
            _edge_pipeline(h_ref, acc, idx3_ref,
                           tid * n_ss, idxb, rows, gsem, ssem, isem, n_ss)
            plsc.subcore_barrier()
            _writeback(acc, out_ref, s, seed_rows, rem_base, rem_rows)

        @pl.when(c == 0)
        def _():
            run(o0_ref)

        @pl.when(c == 1)
        def _():
            run(o1_ref)

    return agg(h, idx3)


def _sc_aggregate(h_lo, h_hi, idx3, e_per_tile, n_ss):
    n, fw = h_lo.shape
    seed_rows = (n // NS) // 8 * 8
    rem_base = seed_rows * NS
    rem_rows = n - rem_base

    mesh = plsc.VectorSubcoreMesh(core_axis_name="c", subcore_axis_name="s")

    @functools.partial(
        pl.kernel,
        out_type=(
            jax.ShapeDtypeStruct((n, fw), jnp.float32),
            jax.ShapeDtypeStruct((n, fw), jnp.float32),
        ),
        mesh=mesh,
        scratch_types=_sc_scratch(n, fw, e_per_tile),
    )
    def agg(h_lo_ref, h_hi_ref, idx3_ref, out_lo_ref, out_hi_ref,
            acc, idxb, rows, gsem, ssem, isem):
        c = lax.axis_index("c")
        s = lax.axis_index("s")

        def run(h_ref, out_ref):
            _seed(h_ref, acc, s, seed_rows, rem_base, rem_rows)
            plsc.subcore_barrier()
            _edge_pipeline(h_ref, acc, idx3_ref,
                           s * n_ss, idxb, rows, gsem, ssem, isem, n_ss)
            plsc.subcore_barrier()
            _writeback(acc, out_ref, s, seed_rows, rem_base, rem_rows)

        @pl.when(c == 0)
        def _():
            run(h_lo_ref, out_lo_ref)

        @pl.when(c == 1)
        def _():
            run(h_hi_ref, out_hi_ref)

    return agg(h_lo, h_hi, idx3)


_ROWS = 2000


def _mlp0(o0, o1, x, W1, b1, W2, b2):
    n, fw = o0.shape
    h = W2.shape[0]

    def body(o0_ref, o1_ref, x_ref, w1_ref, b1_ref, w2_ref, b2_ref,
             out_lo_ref, out_hi_ref):
        hs = o0_ref[...] + o1_ref[...] - x_ref[...]
        z = jnp.dot(hs, w1_ref[...], preferred_element_type=jnp.float32)
        z = jnp.maximum(z + b1_ref[...], 0.0)
        z = jnp.dot(z, w2_ref[...], preferred_element_type=jnp.float32) + b2_ref[...]
        z = jnp.maximum(z, 0.0)
        hw = h // 2
        out_lo_ref[...] = z[:, :hw]
        out_hi_ref[...] = z[:, hw:]

    grid = (n // _ROWS,)
    return pl.pallas_call(
        body,
        grid=grid,
        in_specs=[
            pl.BlockSpec((_ROWS, fw), lambda i: (i, 0)),
            pl.BlockSpec((_ROWS, fw), lambda i: (i, 0)),
            pl.BlockSpec((_ROWS, fw), lambda i: (i, 0)),
            pl.BlockSpec((fw, h), lambda i: (0, 0)),
            pl.BlockSpec((1, h), lambda i: (0, 0)),
            pl.BlockSpec((h, h), lambda i: (0, 0)),
            pl.BlockSpec((1, h), lambda i: (0, 0)),
        ],
        out_specs=[
            pl.BlockSpec((_ROWS, h // 2), lambda i: (i, 0)),
            pl.BlockSpec((_ROWS, h // 2), lambda i: (i, 0)),
        ],
        out_shape=[
            jax.ShapeDtypeStruct((n, h // 2), jnp.float32),
            jax.ShapeDtypeStruct((n, h // 2), jnp.float32),
        ],
    )(o0, o1, x, W1, b1, W2, b2)


def _mlp_res(hs_lo, hs_hi, hin_lo, hin_hi, W1a, W1b, b1, W2, b2):
    n, fw = hs_lo.shape
    h = W2.shape[0]

    def body(lo_ref, hi_ref, ilo_ref, ihi_ref, w1a_ref, w1b_ref, b1_ref,
             w2_ref, b2_ref, out_lo_ref, out_hi_ref):
        z = jnp.dot(lo_ref[...], w1a_ref[...], preferred_element_type=jnp.float32)
        z += jnp.dot(hi_ref[...], w1b_ref[...], preferred_element_type=jnp.float32)
        z = jnp.maximum(z + b1_ref[...], 0.0)
        z = jnp.dot(z, w2_ref[...], preferred_element_type=jnp.float32) + b2_ref[...]
        hw = h // 2
        out_lo_ref[...] = jnp.maximum(z[:, :hw] + lo_ref[...], 0.0) + ilo_ref[...]
        out_hi_ref[...] = jnp.maximum(z[:, hw:] + hi_ref[...], 0.0) + ihi_ref[...]

    grid = (n // _ROWS,)
    return pl.pallas_call(
        body,
        grid=grid,
        in_specs=[
            pl.BlockSpec((_ROWS, fw), lambda i: (i, 0)),
            pl.BlockSpec((_ROWS, fw), lambda i: (i, 0)),
            pl.BlockSpec((_ROWS, fw), lambda i: (i, 0)),
            pl.BlockSpec((_ROWS, fw), lambda i: (i, 0)),
            pl.BlockSpec((fw, h), lambda i: (0, 0)),
            pl.BlockSpec((fw, h), lambda i: (0, 0)),
            pl.BlockSpec((1, h), lambda i: (0, 0)),
            pl.BlockSpec((h, h), lambda i: (0, 0)),
            pl.BlockSpec((1, h), lambda i: (0, 0)),
        ],
        out_specs=[
            pl.BlockSpec((_ROWS, fw), lambda i: (i, 0)),
            pl.BlockSpec((_ROWS, fw), lambda i: (i, 0)),
        ],
        out_shape=[
            jax.ShapeDtypeStruct((n, fw), jnp.float32),
            jax.ShapeDtypeStruct((n, fw), jnp.float32),
        ],
    )(hs_lo, hs_hi, hin_lo, hin_hi, W1a, W1b, b1, W2, b2)


def _mlp_pool(hs_lo, hs_hi, hin_lo, hin_hi, batch2d, W1a, W1b, b1, W2, b2,
              Wc1, bc1, Wc2, bc2, g):
    n, fw = hs_lo.shape
    h = W2.shape[0]
    c = Wc2.shape[1]

    def body(lo_ref, hi_ref, ilo_ref, ihi_ref, b_ref, w1a_ref, w1b_ref, b1_ref,
             w2_ref, b2_ref, wc1_ref, bc1_ref, wc2_ref, bc2_ref, out_ref,
             sums_ref, counts_ref):
        z = jnp.dot(lo_ref[...], w1a_ref[...], preferred_element_type=jnp.float32)
        z += jnp.dot(hi_ref[...], w1b_ref[...], preferred_element_type=jnp.float32)
        z = jnp.maximum(z + b1_ref[...], 0.0)
        z = jnp.dot(z, w2_ref[...], preferred_element_type=jnp.float32) + b2_ref[...]
        hw = h // 2
        o_lo = jnp.maximum(z[:, :hw] + lo_ref[...], 0.0) + ilo_ref[...]
        o_hi = jnp.maximum(z[:, hw:] + hi_ref[...], 0.0) + ihi_ref[...]
        h4 = jnp.concatenate([o_lo, o_hi], axis=1)
        onehot = (b_ref[...] == lax.broadcasted_iota(jnp.int32, (_ROWS, g), 1))
        onehot = onehot.astype(jnp.float32)
        psum = lax.dot_general(onehot, h4, (((0,), (0,)), ((), ())),
                               preferred_element_type=jnp.float32)
        ones = jnp.ones((_ROWS, 8), jnp.float32)
        pcnt = lax.dot_general(onehot, ones, (((0,), (0,)), ((), ())),
                               preferred_element_type=jnp.float32)

        @pl.when(pl.program_id(0) == 0)
        def _():
            sums_ref[...] = jnp.zeros_like(sums_ref)
            counts_ref[...] = jnp.zeros_like(counts_ref)

        sums_ref[...] += psum
        counts_ref[...] += pcnt

        @pl.when(pl.program_id(0) == pl.num_programs(0) - 1)
        def _():
            cnt = jnp.maximum(counts_ref[...][:, :1], 1.0)
            pooled = sums_ref[...] / cnt
            zc = jnp.maximum(
                jnp.dot(pooled, wc1_ref[...],
                        preferred_element_type=jnp.float32) + bc1_ref[...],
                0.0)
            out_ref[...] = (
                jnp.dot(zc, wc2_ref[...], preferred_element_type=jnp.float32)
                + bc2_ref[...])

    grid = (n // _ROWS,)
    return pl.pallas_call(
        body,
        grid=grid,
        in_specs=[
            pl.BlockSpec((_ROWS, fw), lambda i: (i, 0)),
            pl.BlockSpec((_ROWS, fw), lambda i: (i, 0)),
            pl.BlockSpec((_ROWS, fw), lambda i: (i, 0)),
            pl.BlockSpec((_ROWS, fw), lambda i: (i, 0)),
            pl.BlockSpec((_ROWS, 1), lambda i: (i, 0)),
            pl.BlockSpec((fw, h), lambda i: (0, 0)),
            pl.BlockSpec((fw, h), lambda i: (0, 0)),
            pl.BlockSpec((1, h), lambda i: (0, 0)),
            pl.BlockSpec((h, h), lambda i: (0, 0)),
            pl.BlockSpec((1, h), lambda i: (0, 0)),
            pl.BlockSpec((h, h), lambda i: (0, 0)),
            pl.BlockSpec((1, h), lambda i: (0, 0)),
            pl.BlockSpec((h, c), lambda i: (0, 0)),
            pl.BlockSpec((1, c), lambda i: (0, 0)),
        ],
        out_specs=pl.BlockSpec((g, c), lambda i: (0, 0)),
        out_shape=jax.ShapeDtypeStruct((g, c), jnp.float32),
        scratch_shapes=[
            pltpu.VMEM((g, h), jnp.float32),
            pltpu.VMEM((g, 8), jnp.float32),
        ],
    )(hs_lo, hs_hi, hin_lo, hin_hi, batch2d, W1a, W1b, b1, W2, b2,
      Wc1, bc1, Wc2, bc2)


def kernel(x, edge_index, batch, W1_0, b1_0, W2_0, b2_0, Ws1, bs1, Ws2, bs2,
           Wc1, bc1, Wc2, bc2):
    n, d_in = x.shape
    h = W2_0.shape[0]
    g = 64
    src = edge_index[0]
    dst = edge_index[1]
    hw = h // 2

    idx30, ept0, nss0 = _pad_edges(src, dst, 2 * NS, n)
    idx3f, eptf, nssf = _pad_edges(src, dst, NS, n)

    o0, o1 = _sc_aggregate_edges(x, idx30, ept0, nss0)
    h_lo, h_hi = _mlp0(o0, o1, x, W1_0, b1_0.reshape(1, -1), W2_0,
                       b2_0.reshape(1, -1))

    for i in range(Ws1.shape[0]):
        hs_lo, hs_hi = _sc_aggregate(h_lo, h_hi, idx3f, eptf, nssf)
        W1a, W1b = Ws1[i][:hw], Ws1[i][hw:]
        b1i = bs1[i].reshape(1, -1)
        W2i, b2i = Ws2[i], bs2[i].reshape(1, -1)
        if i < Ws1.shape[0] - 1:
            h_lo, h_hi = _mlp_res(hs_lo, hs_hi, h_lo, h_hi, W1a, W1b, b1i,
                                  W2i, b2i)
        else:
            logits = _mlp_pool(hs_lo, hs_hi, h_lo, h_hi,
                               batch.reshape(n, 1), W1a, W1b, b1i, W2i, b2i,
                               Wc1, bc1.reshape(1, -1), Wc2,
                               bc2.reshape(1, -1), g)

    return logits


# speedup vs baseline: 1.0121x; 1.0086x over previous
import functools

import jax
import jax.numpy as jnp
from jax import lax
from jax.experimental import pallas as pl
from jax.experimental.pallas import tpu as pltpu
from jax.experimental.pallas import tpu_sc as plsc

NS = 16
W_EDGES = 64
NBUF = 5
PAD_SPREAD = 8


def _pad_edges(src, dst, n_chunks, n):
    e = src.shape[0]
    per = e // n_chunks
    padded = -(-per // (NBUF * W_EDGES)) * (NBUF * W_EDGES)
    pad = padded - per
    pad_src = (jnp.arange(pad, dtype=jnp.int32) * 97) % n
    pad_dst = n + (jnp.arange(pad, dtype=jnp.int32) % PAD_SPREAD)
    srcp = jnp.concatenate(
        [src.reshape(n_chunks, per),
         jnp.broadcast_to(pad_src, (n_chunks, pad))], axis=1)
    dstp = jnp.concatenate(
        [dst.reshape(n_chunks, per),
         jnp.broadcast_to(pad_dst, (n_chunks, pad))], axis=1)
    n_ss = padded // (NBUF * W_EDGES)
    s4 = srcp.reshape(n_chunks, n_ss, NBUF, 1, W_EDGES)
    d4 = dstp.reshape(n_chunks, n_ss, NBUF, 1, W_EDGES)
    idx3 = jnp.concatenate([s4, d4], axis=3).reshape(
        n_chunks * n_ss, NBUF * 2, W_EDGES)
    return idx3, padded, n_ss


def _edge_pipeline(h_ref, acc, idx3_ref, idx_row_base,
                   idxb, rows, gsem, ssem, isem, n_ss):
    def idx_args(ss, p):
        return (idx3_ref.at[idx_row_base + ss], idxb.at[p], isem.at[p])

    def g_args(p, b):
        return (h_ref.at[idxb.at[p].at[2 * b]], rows.at[b], gsem.at[b])

    def s_args(p, b):
        return (rows.at[b], acc.at[idxb.at[p].at[2 * b + 1]], ssem.at[b])

    def do_ss(ss, p, prefetch_ss):
        if prefetch_ss is not None:
            pltpu.async_copy(*idx_args(prefetch_ss, 1 - p))
        for b in range(NBUF):
            pltpu.make_async_copy(*g_args(p, b)).wait()
            pltpu.async_copy(*s_args(p, b), add=True)
        if prefetch_ss is not None:
            pltpu.make_async_copy(*idx_args(prefetch_ss, 1 - p)).wait()
        for b in range(NBUF):
            pltpu.make_async_copy(*s_args(p, b)).wait()
            if prefetch_ss is not None:
                pltpu.async_copy(*g_args(1 - p, b))

    pro = idx_args(0, 0)
    pltpu.sync_copy(pro[0], pro[1])
    for b in range(NBUF):
        pltpu.async_copy(*g_args(0, b))

    n_double = (n_ss - 1) // 2

    def body(i2, carry):
        do_ss(2 * i2, 0, 2 * i2 + 1)
        do_ss(2 * i2 + 1, 1, 2 * i2 + 2)
        return carry

    lax.fori_loop(0, n_double, body, 0)
    for ss in range(2 * n_double, n_ss):
        do_ss(ss, ss % 2, ss + 1 if ss < n_ss - 1 else None)

def _seed(h_ref, acc, s, seed_rows, rem_base, rem_rows):
    pltpu.sync_copy(h_ref.at[pl.ds(s * seed_rows, seed_rows)],
                    acc.at[pl.ds(s * seed_rows, seed_rows)])

    @pl.when(s == NS - 1)
    def _():
        pltpu.sync_copy(h_ref.at[pl.ds(rem_base, rem_rows)],
                        acc.at[pl.ds(rem_base, rem_rows)])


def _writeback(acc, out_ref, s, seed_rows, rem_base, rem_rows):
    pltpu.sync_copy(acc.at[pl.ds(s * seed_rows, seed_rows)],
                    out_ref.at[pl.ds(s * seed_rows, seed_rows)])

    @pl.when(s == NS - 1)
    def _():
        pltpu.sync_copy(acc.at[pl.ds(rem_base, rem_rows)],
                        out_ref.at[pl.ds(rem_base, rem_rows)])


def _sc_scratch(n, fw, e_per_tile):
    return (
        pltpu.VMEM_SHARED((n + PAD_SPREAD, fw), jnp.float32),
        pltpu.VMEM((2, NBUF * 2, W_EDGES), jnp.int32),
        pltpu.VMEM((NBUF, W_EDGES, fw), jnp.float32),
        pltpu.SemaphoreType.DMA((NBUF,)),
        pltpu.SemaphoreType.DMA((NBUF,)),
        pltpu.SemaphoreType.DMA((2,)),
    )


def _sc_aggregate_edges(h, idx3, e_per_tile, n_ss):
    n, fw = h.shape
    seed_rows = (n // NS) // 8 * 8
    rem_base = seed_rows * NS
    rem_rows = n - rem_base

    mesh = plsc.VectorSubcoreMesh(core_axis_name="c", subcore_axis_name="s")

    @functools.partial(
        pl.kernel,
        out_type=(
            jax.ShapeDtypeStruct((n, fw), jnp.float32),
            jax.ShapeDtypeStruct((n, fw), jnp.float32),
        ),
        mesh=mesh,
        scratch_types=_sc_scratch(n, fw, e_per_tile),
    )
    def agg(h_ref, idx3_ref, o0_ref, o1_ref, acc, idxb, rows,
            gsem, ssem, isem):
        c = lax.axis_index("c")
        s = lax.axis_index("s")
        tid = c * NS + s

        def run(out_ref):
            _seed(h_ref, acc, s, seed_rows, rem_base, rem_rows)
            plsc.subcore_barrier()
            _edge_pipeline(h_ref, acc, idx3_ref,
                           tid * n_ss, idxb, rows, gsem, ssem, isem, n_ss)
            plsc.subcore_barrier()
            _writeback(acc, out_ref, s, seed_rows, rem_base, rem_rows)

        @pl.when(c == 0)
        def _():
            run(o0_ref)

        @pl.when(c == 1)
        def _():
            run(o1_ref)

    return agg(h, idx3)


def _sc_aggregate(h_lo, h_hi, idx3, e_per_tile, n_ss):
    n, fw = h_lo.shape
    seed_rows = (n // NS) // 8 * 8
    rem_base = seed_rows * NS
    rem_rows = n - rem_base

    mesh = plsc.VectorSubcoreMesh(core_axis_name="c", subcore_axis_name="s")

    @functools.partial(
        pl.kernel,
        out_type=(
            jax.ShapeDtypeStruct((n, fw), jnp.float32),
            jax.ShapeDtypeStruct((n, fw), jnp.float32),
        ),
        mesh=mesh,
        scratch_types=_sc_scratch(n, fw, e_per_tile),
    )
    def agg(h_lo_ref, h_hi_ref, idx3_ref, out_lo_ref, out_hi_ref,
            acc, idxb, rows, gsem, ssem, isem):
        c = lax.axis_index("c")
        s = lax.axis_index("s")

        def run(h_ref, out_ref):
            _seed(h_ref, acc, s, seed_rows, rem_base, rem_rows)
            plsc.subcore_barrier()
            _edge_pipeline(h_ref, acc, idx3_ref,
                           s * n_ss, idxb, rows, gsem, ssem, isem, n_ss)
            plsc.subcore_barrier()
            _writeback(acc, out_ref, s, seed_rows, rem_base, rem_rows)

        @pl.when(c == 0)
        def _():
            run(h_lo_ref, out_lo_ref)

        @pl.when(c == 1)
        def _():
            run(h_hi_ref, out_hi_ref)

    return agg(h_lo, h_hi, idx3)


_ROWS = 5000


def _mlp0(o0, o1, x, W1, b1, W2, b2):
    n, fw = o0.shape
    h = W2.shape[0]

    def body(o0_ref, o1_ref, x_ref, w1_ref, b1_ref, w2_ref, b2_ref,
             out_lo_ref, out_hi_ref):
        hs = o0_ref[...] + o1_ref[...] - x_ref[...]
        z = jnp.dot(hs, w1_ref[...], preferred_element_type=jnp.float32)
        z = jnp.maximum(z + b1_ref[...], 0.0)
        z = jnp.dot(z, w2_ref[...], preferred_element_type=jnp.float32) + b2_ref[...]
        z = jnp.maximum(z, 0.0)
        hw = h // 2
        out_lo_ref[...] = z[:, :hw]
        out_hi_ref[...] = z[:, hw:]

    grid = (n // _ROWS,)
    return pl.pallas_call(
        body,
        grid=grid,
        in_specs=[
            pl.BlockSpec((_ROWS, fw), lambda i: (i, 0)),
            pl.BlockSpec((_ROWS, fw), lambda i: (i, 0)),
            pl.BlockSpec((_ROWS, fw), lambda i: (i, 0)),
            pl.BlockSpec((fw, h), lambda i: (0, 0)),
            pl.BlockSpec((1, h), lambda i: (0, 0)),
            pl.BlockSpec((h, h), lambda i: (0, 0)),
            pl.BlockSpec((1, h), lambda i: (0, 0)),
        ],
        out_specs=[
            pl.BlockSpec((_ROWS, h // 2), lambda i: (i, 0)),
            pl.BlockSpec((_ROWS, h // 2), lambda i: (i, 0)),
        ],
        out_shape=[
            jax.ShapeDtypeStruct((n, h // 2), jnp.float32),
            jax.ShapeDtypeStruct((n, h // 2), jnp.float32),
        ],
    )(o0, o1, x, W1, b1, W2, b2)


def _mlp_res(hs_lo, hs_hi, hin_lo, hin_hi, W1a, W1b, b1, W2, b2):
    n, fw = hs_lo.shape
    h = W2.shape[0]

    def body(lo_ref, hi_ref, ilo_ref, ihi_ref, w1a_ref, w1b_ref, b1_ref,
             w2_ref, b2_ref, out_lo_ref, out_hi_ref):
        z = jnp.dot(lo_ref[...], w1a_ref[...], preferred_element_type=jnp.float32)
        z += jnp.dot(hi_ref[...], w1b_ref[...], preferred_element_type=jnp.float32)
        z = jnp.maximum(z + b1_ref[...], 0.0)
        z = jnp.dot(z, w2_ref[...], preferred_element_type=jnp.float32) + b2_ref[...]
        hw = h // 2
        out_lo_ref[...] = jnp.maximum(z[:, :hw] + lo_ref[...], 0.0) + ilo_ref[...]
        out_hi_ref[...] = jnp.maximum(z[:, hw:] + hi_ref[...], 0.0) + ihi_ref[...]

    grid = (n // _ROWS,)
    return pl.pallas_call(
        body,
        grid=grid,
        in_specs=[
            pl.BlockSpec((_ROWS, fw), lambda i: (i, 0)),
            pl.BlockSpec((_ROWS, fw), lambda i: (i, 0)),
            pl.BlockSpec((_ROWS, fw), lambda i: (i, 0)),
            pl.BlockSpec((_ROWS, fw), lambda i: (i, 0)),
            pl.BlockSpec((fw, h), lambda i: (0, 0)),
            pl.BlockSpec((fw, h), lambda i: (0, 0)),
            pl.BlockSpec((1, h), lambda i: (0, 0)),
            pl.BlockSpec((h, h), lambda i: (0, 0)),
            pl.BlockSpec((1, h), lambda i: (0, 0)),
        ],
        out_specs=[
            pl.BlockSpec((_ROWS, fw), lambda i: (i, 0)),
            pl.BlockSpec((_ROWS, fw), lambda i: (i, 0)),
        ],
        out_shape=[
            jax.ShapeDtypeStruct((n, fw), jnp.float32),
            jax.ShapeDtypeStruct((n, fw), jnp.float32),
        ],
    )(hs_lo, hs_hi, hin_lo, hin_hi, W1a, W1b, b1, W2, b2)


def _mlp_pool(hs_lo, hs_hi, hin_lo, hin_hi, batch2d, W1a, W1b, b1, W2, b2,
              Wc1, bc1, Wc2, bc2, g):
    n, fw = hs_lo.shape
    h = W2.shape[0]
    c = Wc2.shape[1]

    def body(lo_ref, hi_ref, ilo_ref, ihi_ref, b_ref, w1a_ref, w1b_ref, b1_ref,
             w2_ref, b2_ref, wc1_ref, bc1_ref, wc2_ref, bc2_ref, out_ref,
             sums_ref, counts_ref):
        z = jnp.dot(lo_ref[...], w1a_ref[...], preferred_element_type=jnp.float32)
        z += jnp.dot(hi_ref[...], w1b_ref[...], preferred_element_type=jnp.float32)
        z = jnp.maximum(z + b1_ref[...], 0.0)
        z = jnp.dot(z, w2_ref[...], preferred_element_type=jnp.float32) + b2_ref[...]
        hw = h // 2
        o_lo = jnp.maximum(z[:, :hw] + lo_ref[...], 0.0) + ilo_ref[...]
        o_hi = jnp.maximum(z[:, hw:] + hi_ref[...], 0.0) + ihi_ref[...]
        h4 = jnp.concatenate([o_lo, o_hi], axis=1)
        onehot = (b_ref[...] == lax.broadcasted_iota(jnp.int32, (_ROWS, g), 1))
        onehot = onehot.astype(jnp.float32)
        psum = lax.dot_general(onehot, h4, (((0,), (0,)), ((), ())),
                               preferred_element_type=jnp.float32)
        ones = jnp.ones((_ROWS, 8), jnp.float32)
        pcnt = lax.dot_general(onehot, ones, (((0,), (0,)), ((), ())),
                               preferred_element_type=jnp.float32)

        @pl.when(pl.program_id(0) == 0)
        def _():
            sums_ref[...] = jnp.zeros_like(sums_ref)
            counts_ref[...] = jnp.zeros_like(counts_ref)

        sums_ref[...] += psum
        counts_ref[...] += pcnt

        @pl.when(pl.program_id(0) == pl.num_programs(0) - 1)
        def _():
            cnt = jnp.maximum(counts_ref[...][:, :1], 1.0)
            pooled = sums_ref[...] / cnt
            zc = jnp.maximum(
                jnp.dot(pooled, wc1_ref[...],
                        preferred_element_type=jnp.float32) + bc1_ref[...],
                0.0)
            out_ref[...] = (
                jnp.dot(zc, wc2_ref[...], preferred_element_type=jnp.float32)
                + bc2_ref[...])

    grid = (n // _ROWS,)
    return pl.pallas_call(
        body,
        grid=grid,
        in_specs=[
            pl.BlockSpec((_ROWS, fw), lambda i: (i, 0)),
            pl.BlockSpec((_ROWS, fw), lambda i: (i, 0)),
            pl.BlockSpec((_ROWS, fw), lambda i: (i, 0)),
            pl.BlockSpec((_ROWS, fw), lambda i: (i, 0)),
            pl.BlockSpec((_ROWS, 1), lambda i: (i, 0)),
            pl.BlockSpec((fw, h), lambda i: (0, 0)),
            pl.BlockSpec((fw, h), lambda i: (0, 0)),
            pl.BlockSpec((1, h), lambda i: (0, 0)),
            pl.BlockSpec((h, h), lambda i: (0, 0)),
            pl.BlockSpec((1, h), lambda i: (0, 0)),
            pl.BlockSpec((h, h), lambda i: (0, 0)),
            pl.BlockSpec((1, h), lambda i: (0, 0)),
            pl.BlockSpec((h, c), lambda i: (0, 0)),
            pl.BlockSpec((1, c), lambda i: (0, 0)),
        ],
        out_specs=pl.BlockSpec((g, c), lambda i: (0, 0)),
        out_shape=jax.ShapeDtypeStruct((g, c), jnp.float32),
        scratch_shapes=[
            pltpu.VMEM((g, h), jnp.float32),
            pltpu.VMEM((g, 8), jnp.float32),
        ],
    )(hs_lo, hs_hi, hin_lo, hin_hi, batch2d, W1a, W1b, b1, W2, b2,
      Wc1, bc1, Wc2, bc2)


def kernel(x, edge_index, batch, W1_0, b1_0, W2_0, b2_0, Ws1, bs1, Ws2, bs2,
           Wc1, bc1, Wc2, bc2):
    n, d_in = x.shape
    h = W2_0.shape[0]
    g = 64
    src = edge_index[0]
    dst = edge_index[1]
    hw = h // 2

    idx30, ept0, nss0 = _pad_edges(src, dst, 2 * NS, n)
    idx3f, eptf, nssf = _pad_edges(src, dst, NS, n)

    o0, o1 = _sc_aggregate_edges(x, idx30, ept0, nss0)
    h_lo, h_hi = _mlp0(o0, o1, x, W1_0, b1_0.reshape(1, -1), W2_0,
                       b2_0.reshape(1, -1))

    for i in range(Ws1.shape[0]):
        hs_lo, hs_hi = _sc_aggregate(h_lo, h_hi, idx3f, eptf, nssf)
        W1a, W1b = Ws1[i][:hw], Ws1[i][hw:]
        b1i = bs1[i].reshape(1, -1)
        W2i, b2i = Ws2[i], bs2[i].reshape(1, -1)
        if i < Ws1.shape[0] - 1:
            h_lo, h_hi = _mlp_res(hs_lo, hs_hi, h_lo, h_hi, W1a, W1b, b1i,
                                  W2i, b2i)
        else:
            logits = _mlp_pool(hs_lo, hs_hi, h_lo, h_hi,
                               batch.reshape(n, 1), W1a, W1b, b1i, W2i, b2i,
                               Wc1, bc1.reshape(1, -1), Wc2,
                               bc2.reshape(1, -1), g)

    return logits
